# baseline (speedup 1.0000x reference)
import functools

import jax
import jax.numpy as jnp
from jax import lax
from jax.experimental import pallas as pl
from jax.experimental.pallas import tpu as pltpu
from jax.experimental.pallas import tpu_sc as plsc

N_NODES = 10000
N_EDGES = 320000
D = 128
NP = 10240
NW = 32
GP = 128
NG = 2560
E_PAD = NG * GP
T = NG // NW
PAIRS = T // 2
RPT = NP // 16

_MESH = plsc.VectorSubcoreMesh(core_axis_name="c", subcore_axis_name="s")


KB = 8
NB = T // KB
BPAIRS = NB // 2


def _sc_aggregate(xe, src2, dst2, z2d):
    @functools.partial(
        pl.kernel,
        mesh=_MESH,
        out_type=jax.ShapeDtypeStruct((2, NP, D), jnp.float32),
        scratch_types=[
            pltpu.VMEM((KB, GP), jnp.int32),
            pltpu.VMEM((KB, GP), jnp.int32),
            pltpu.VMEM((KB, GP), jnp.int32),
            pltpu.VMEM((KB, GP), jnp.int32),
            pltpu.VMEM((GP, D), jnp.float32),
            pltpu.VMEM_SHARED((NP, D), jnp.float32),
            pltpu.SemaphoreType.DMA,
            pltpu.SemaphoreType.DMA,
            pltpu.SemaphoreType.DMA,
        ],
    )
    def agg(xe_hbm, src_hbm, dst_hbm, z2d_hbm, out_hbm,
            sidxA, didxA, sidxB, didxB, rows, acc, isemA, isemB, gsem):
        core = lax.axis_index("c")
        tid = lax.axis_index("s")
        w = core * 16 + tid

        pltpu.sync_copy(z2d_hbm, rows)
        rbase = tid * RPT
        for i in range(RPT // GP):
            pltpu.sync_copy(rows, acc.at[pl.ds(rbase + i * GP, GP)])
        plsc.subcore_barrier()

        def start_idx(b, sidx, didx, isem):
            row = (w * NB + jnp.minimum(b, NB - 1)) * KB
            pltpu.async_copy(src_hbm.at[pl.ds(row, KB)], sidx, isem)
            pltpu.async_copy(dst_hbm.at[pl.ds(row, KB)], didx, isem)

        def wait_idx(sidx, didx, isem):
            pltpu.make_async_copy(src_hbm.at[pl.ds(0, KB)], sidx, isem).wait()
            pltpu.make_async_copy(dst_hbm.at[pl.ds(0, KB)], didx, isem).wait()

        def run_batch(sidx, didx):
            for j in range(KB):
                pltpu.async_copy(xe_hbm.at[sidx.at[j]], rows, gsem).wait()
                pltpu.sync_copy(rows, acc.at[didx.at[j]], add=True)

        start_idx(0, sidxA, didxA, isemA)

        def body(p, carry):
            b = 2 * p
            wait_idx(sidxA, didxA, isemA)
            start_idx(b + 1, sidxB, didxB, isemB)
            run_batch(sidxA, didxA)
            wait_idx(sidxB, didxB, isemB)
            start_idx(b + 2, sidxA, didxA, isemA)
            run_batch(sidxB, didxB)
            return carry

        lax.fori_loop(0, BPAIRS, body, 0)
        wait_idx(sidxA, didxA, isemA)

        plsc.subcore_barrier()

        for i in range(RPT // GP):
            pltpu.sync_copy(acc.at[pl.ds(rbase + i * GP, GP)], rows)
            pltpu.sync_copy(rows, out_hbm.at[core, pl.ds(rbase + i * GP, GP)])

    return agg(xe, src2, dst2, z2d)


def _sc_count(dst, z2d, o2d):
    @functools.partial(
        pl.kernel,
        mesh=_MESH,
        out_type=jax.ShapeDtypeStruct((2, NP, D), jnp.float32),
        scratch_types=[
            pltpu.VMEM((GP,), jnp.int32),
            pltpu.VMEM((GP,), jnp.int32),
            pltpu.VMEM((GP, D), jnp.float32),
            pltpu.VMEM_SHARED((NP, D), jnp.float32),
            pltpu.SemaphoreType.DMA,
            pltpu.SemaphoreType.DMA,
        ],
    )
    def cnt_k(dst_hbm, z2d_hbm, o2d_hbm, out_hbm, didxA, didxB, rows, acc,
              isemA, isemB):
        core = lax.axis_index("c")
        tid = lax.axis_index("s")
        w = core * 16 + tid

        pltpu.sync_copy(z2d_hbm, rows)
        rbase = tid * RPT
        for i in range(RPT // GP):
            pltpu.sync_copy(rows, acc.at[pl.ds(rbase + i * GP, GP)])
        plsc.subcore_barrier()

        pltpu.sync_copy(o2d_hbm, rows)

        def base_of(k):
            return jnp.minimum(k * NW + w, NG - 1) * GP

        def start_idx(k, didx, isem):
            pltpu.async_copy(dst_hbm.at[pl.ds(base_of(k), GP)], didx, isem)

        def wait_idx(didx, isem):
            pltpu.make_async_copy(dst_hbm.at[pl.ds(0, GP)], didx, isem).wait()

        start_idx(0, didxA, isemA)

        def body(p, carry):
            k = 2 * p
            wait_idx(didxA, isemA)
            start_idx(k + 1, didxB, isemB)
            pltpu.sync_copy(rows, acc.at[didxA], add=True)
            wait_idx(didxB, isemB)
            start_idx(k + 2, didxA, isemA)
            pltpu.sync_copy(rows, acc.at[didxB], add=True)
            return carry

        lax.fori_loop(0, PAIRS, body, 0)
        wait_idx(didxA, isemA)

        plsc.subcore_barrier()

        for i in range(RPT // GP):
            pltpu.sync_copy(acc.at[pl.ds(rbase + i * GP, GP)], rows)
            pltpu.sync_copy(rows, out_hbm.at[core, pl.ds(rbase + i * GP, GP)])

    return cnt_k(dst, z2d, o2d)


def _tc_dense(xe, agg_part, cnt_part, W_l, W_r, b, relu):
    B = 1280

    def body(x_ref, a_ref, c_ref, wl_ref, wr_ref, b_ref, o_ref):
        a = a_ref[0] + a_ref[1]
        c = c_ref[0] + c_ref[1]
        mean = a / jnp.maximum(c, 1.0)
        dn = (((1,), (1,)), ((), ()))
        out = (lax.dot_general(mean, wl_ref[...], dn,
                               preferred_element_type=jnp.float32)
               + lax.dot_general(x_ref[...], wr_ref[...], dn,
                                 preferred_element_type=jnp.float32)
               + b_ref[...][None, :])
        if relu:
            out = jnp.maximum(out, 0.0)
        o_ref[...] = out

    return pl.pallas_call(
        body,
        grid=(NP // B,),
        in_specs=[
            pl.BlockSpec((B, D), lambda i: (i, 0)),
            pl.BlockSpec((2, B, D), lambda i: (0, i, 0)),
            pl.BlockSpec((2, B, D), lambda i: (0, i, 0)),
            pl.BlockSpec((D, D), lambda i: (0, 0)),
            pl.BlockSpec((D, D), lambda i: (0, 0)),
            pl.BlockSpec((D,), lambda i: (0,)),
        ],
        out_specs=pl.BlockSpec((B, D), lambda i: (i, 0)),
        out_shape=jax.ShapeDtypeStruct((NP, D), jnp.float32),
    )(xe, agg_part, cnt_part, W_l, W_r, b)


def kernel(x, edge_index, W1_l, W1_r, b1, W2_l, W2_r, b2):
    src = edge_index[0]
    dst = edge_index[1]

    npad = E_PAD - N_EDGES
    srcp = jnp.concatenate([src, jnp.zeros((npad,), jnp.int32)])
    dstp = jnp.concatenate([dst, jnp.full((npad,), N_NODES, jnp.int32)])
    src2 = srcp.reshape(NG, GP)
    dst2 = dstp.reshape(NG, GP)

    xe = jnp.pad(x, ((0, NP - N_NODES), (0, 0)))
    z2d = jnp.zeros((GP, D), jnp.float32)
    o2d = jnp.ones((GP, D), jnp.float32)

    cnt = _sc_count(dstp, z2d, o2d)
    agg1 = _sc_aggregate(xe, src2, dst2, z2d)
    h = _tc_dense(xe, agg1, cnt, W1_l, W1_r, b1, relu=True)
    agg2 = _sc_aggregate(h, src2, dst2, z2d)
    out = _tc_dense(h, agg2, cnt, W2_l, W2_r, b2, relu=False)
    return out[:N_NODES]


# speedup vs baseline: 1.0848x; 1.0848x over previous
import functools

import jax
import jax.numpy as jnp
from jax import lax
from jax.experimental import pallas as pl
from jax.experimental.pallas import tpu as pltpu
from jax.experimental.pallas import tpu_sc as plsc

N_NODES = 10000
N_EDGES = 320000
D = 128
NP = 10240
NW = 32
GP = 128
NG = 2560
E_PAD = NG * GP
T = NG // NW
PAIRS = T // 2
RPT = NP // 16

_MESH = plsc.VectorSubcoreMesh(core_axis_name="c", subcore_axis_name="s")


KB = 8
NB = T // KB
BPAIRS = NB // 2


def _sc_aggregate(xe, src, dst, z2d):
    @functools.partial(
        pl.kernel,
        mesh=_MESH,
        out_type=jax.ShapeDtypeStruct((2, NP, D), jnp.float32),
        scratch_types=[
            pltpu.VMEM((GP,), jnp.int32),
            pltpu.VMEM((GP,), jnp.int32),
            pltpu.VMEM((GP, D), jnp.float32),
            pltpu.VMEM_SHARED((NP, D), jnp.float32),
            pltpu.SemaphoreType.DMA,
        ],
    )
    def agg(xe_hbm, src_hbm, dst_hbm, z2d_hbm, out_hbm, sidx, didx, rows, acc, sem):
        core = lax.axis_index("c")
        tid = lax.axis_index("s")
        w = core * 16 + tid

        pltpu.sync_copy(z2d_hbm, rows)
        rbase = tid * RPT
        for i in range(RPT // GP):
            pltpu.sync_copy(rows, acc.at[pl.ds(rbase + i * GP, GP)])
        plsc.subcore_barrier()

        def body(g, carry):
            base = (g * NW + w) * GP
            pltpu.sync_copy(src_hbm.at[pl.ds(base, GP)], sidx)
            pltpu.sync_copy(dst_hbm.at[pl.ds(base, GP)], didx)
            pltpu.async_copy(xe_hbm.at[sidx], rows, sem).wait()
            pltpu.sync_copy(rows, acc.at[didx], add=True)
            return carry

        lax.fori_loop(0, T, body, 0)

        plsc.subcore_barrier()

        for i in range(RPT // GP):
            pltpu.sync_copy(acc.at[pl.ds(rbase + i * GP, GP)], rows)
            pltpu.sync_copy(rows, out_hbm.at[core, pl.ds(rbase + i * GP, GP)])

    return agg(xe, src, dst, z2d)


def _sc_count(dst, z2d, o2d):
    @functools.partial(
        pl.kernel,
        mesh=_MESH,
        out_type=jax.ShapeDtypeStruct((2, NP, D), jnp.float32),
        scratch_types=[
            pltpu.VMEM((GP,), jnp.int32),
            pltpu.VMEM((GP,), jnp.int32),
            pltpu.VMEM((GP, D), jnp.float32),
            pltpu.VMEM_SHARED((NP, D), jnp.float32),
            pltpu.SemaphoreType.DMA,
            pltpu.SemaphoreType.DMA,
        ],
    )
    def cnt_k(dst_hbm, z2d_hbm, o2d_hbm, out_hbm, didxA, didxB, rows, acc,
              isemA, isemB):
        core = lax.axis_index("c")
        tid = lax.axis_index("s")
        w = core * 16 + tid

        pltpu.sync_copy(z2d_hbm, rows)
        rbase = tid * RPT
        for i in range(RPT // GP):
            pltpu.sync_copy(rows, acc.at[pl.ds(rbase + i * GP, GP)])
        plsc.subcore_barrier()

        pltpu.sync_copy(o2d_hbm, rows)

        def base_of(k):
            return jnp.minimum(k * NW + w, NG - 1) * GP

        def start_idx(k, didx, isem):
            pltpu.async_copy(dst_hbm.at[pl.ds(base_of(k), GP)], didx, isem)

        def wait_idx(didx, isem):
            pltpu.make_async_copy(dst_hbm.at[pl.ds(0, GP)], didx, isem).wait()

        start_idx(0, didxA, isemA)

        def body(p, carry):
            k = 2 * p
            wait_idx(didxA, isemA)
            start_idx(k + 1, didxB, isemB)
            pltpu.sync_copy(rows, acc.at[didxA], add=True)
            wait_idx(didxB, isemB)
            start_idx(k + 2, didxA, isemA)
            pltpu.sync_copy(rows, acc.at[didxB], add=True)
            return carry

        lax.fori_loop(0, PAIRS, body, 0)
        wait_idx(didxA, isemA)

        plsc.subcore_barrier()

        for i in range(RPT // GP):
            pltpu.sync_copy(acc.at[pl.ds(rbase + i * GP, GP)], rows)
            pltpu.sync_copy(rows, out_hbm.at[core, pl.ds(rbase + i * GP, GP)])

    return cnt_k(dst, z2d, o2d)


def _tc_dense(xe, agg_part, cnt_part, W_l, W_r, b, relu):
    B = 1280

    def body(x_ref, a_ref, c_ref, wl_ref, wr_ref, b_ref, o_ref):
        a = a_ref[0] + a_ref[1]
        c = c_ref[0] + c_ref[1]
        mean = a / jnp.maximum(c, 1.0)
        dn = (((1,), (1,)), ((), ()))
        out = (lax.dot_general(mean, wl_ref[...], dn,
                               preferred_element_type=jnp.float32)
               + lax.dot_general(x_ref[...], wr_ref[...], dn,
                                 preferred_element_type=jnp.float32)
               + b_ref[...][None, :])
        if relu:
            out = jnp.maximum(out, 0.0)
        o_ref[...] = out

    return pl.pallas_call(
        body,
        grid=(NP // B,),
        in_specs=[
            pl.BlockSpec((B, D), lambda i: (i, 0)),
            pl.BlockSpec((2, B, D), lambda i: (0, i, 0)),
            pl.BlockSpec((2, B, D), lambda i: (0, i, 0)),
            pl.BlockSpec((D, D), lambda i: (0, 0)),
            pl.BlockSpec((D, D), lambda i: (0, 0)),
            pl.BlockSpec((D,), lambda i: (0,)),
        ],
        out_specs=pl.BlockSpec((B, D), lambda i: (i, 0)),
        out_shape=jax.ShapeDtypeStruct((NP, D), jnp.float32),
    )(xe, agg_part, cnt_part, W_l, W_r, b)


def kernel(x, edge_index, W1_l, W1_r, b1, W2_l, W2_r, b2):
    src = edge_index[0]
    dst = edge_index[1]

    npad = E_PAD - N_EDGES
    srcp = jnp.concatenate([src, jnp.zeros((npad,), jnp.int32)])
    dstp = jnp.concatenate([dst, jnp.full((npad,), N_NODES, jnp.int32)])
    xe = jnp.pad(x, ((0, NP - N_NODES), (0, 0)))
    z2d = jnp.zeros((GP, D), jnp.float32)
    o2d = jnp.ones((GP, D), jnp.float32)

    cnt = _sc_count(dstp, z2d, o2d)
    agg1 = _sc_aggregate(xe, srcp, dstp, z2d)
    h = _tc_dense(xe, agg1, cnt, W1_l, W1_r, b1, relu=True)
    agg2 = _sc_aggregate(h, srcp, dstp, z2d)
    out = _tc_dense(h, agg2, cnt, W2_l, W2_r, b2, relu=False)
    return out[:N_NODES]


# speedup vs baseline: 2.0338x; 1.8749x over previous
import functools

import jax
import jax.numpy as jnp
from jax import lax
from jax.experimental import pallas as pl
from jax.experimental.pallas import tpu as pltpu
from jax.experimental.pallas import tpu_sc as plsc

N_NODES = 10000
N_EDGES = 320000
D = 128
NP = 10240
NW = 32
GP = 128
NG = 2560
E_PAD = NG * GP
T = NG // NW
PAIRS = T // 2
RPT = NP // 16

_MESH = plsc.VectorSubcoreMesh(core_axis_name="c", subcore_axis_name="s")


KB = 8
NB = T // KB
BPAIRS = NB // 2


def _sc_aggregate(xe, src, dst, z2d):
    @functools.partial(
        pl.kernel,
        mesh=_MESH,
        out_type=jax.ShapeDtypeStruct((2, NP, D), jnp.float32),
        scratch_types=[
            pltpu.VMEM((GP,), jnp.int32),
            pltpu.VMEM((GP,), jnp.int32),
            pltpu.VMEM((GP, D), jnp.float32),
            pltpu.VMEM_SHARED((NP, D), jnp.float32),
            pltpu.SemaphoreType.DMA,
        ],
    )
    def agg(xe_hbm, src_hbm, dst_hbm, z2d_hbm, out_hbm, sidx, didx, rows, acc, sem):
        core = lax.axis_index("c")
        tid = lax.axis_index("s")
        w = core * 16 + tid

        pltpu.sync_copy(z2d_hbm, rows)
        rbase = tid * RPT
        for i in range(RPT // GP):
            pltpu.sync_copy(rows, acc.at[pl.ds(rbase + i * GP, GP)])
        plsc.subcore_barrier()

        def body(g, carry):
            base = (g * NW + w) * GP
            pltpu.sync_copy(src_hbm.at[pl.ds(base, GP)], sidx)
            pltpu.sync_copy(dst_hbm.at[pl.ds(base, GP)], didx)
            pltpu.async_copy(xe_hbm.at[sidx], rows, sem).wait()
            pltpu.sync_copy(rows, acc.at[didx], add=True)
            return carry

        lax.fori_loop(0, T, body, 0)

        plsc.subcore_barrier()

        for i in range(RPT // GP):
            pltpu.sync_copy(acc.at[pl.ds(rbase + i * GP, GP)], rows)
            pltpu.sync_copy(rows, out_hbm.at[core, pl.ds(rbase + i * GP, GP)])

    return agg(xe, src, dst, z2d)


def _sc_count(dst, z2d, o2d):
    @functools.partial(
        pl.kernel,
        mesh=_MESH,
        out_type=jax.ShapeDtypeStruct((2, NP, D), jnp.float32),
        scratch_types=[
            pltpu.VMEM((GP,), jnp.int32),
            pltpu.VMEM((GP,), jnp.int32),
            pltpu.VMEM((GP, D), jnp.float32),
            pltpu.VMEM_SHARED((NP, D), jnp.float32),
            pltpu.SemaphoreType.DMA,
            pltpu.SemaphoreType.DMA,
        ],
    )
    def cnt_k(dst_hbm, z2d_hbm, o2d_hbm, out_hbm, didxA, didxB, rows, acc,
              isemA, isemB):
        core = lax.axis_index("c")
        tid = lax.axis_index("s")
        w = core * 16 + tid

        pltpu.sync_copy(z2d_hbm, rows)
        rbase = tid * RPT
        for i in range(RPT // GP):
            pltpu.sync_copy(rows, acc.at[pl.ds(rbase + i * GP, GP)])
        plsc.subcore_barrier()

        pltpu.sync_copy(o2d_hbm, rows)

        def base_of(k):
            return jnp.minimum(k * NW + w, NG - 1) * GP

        def start_idx(k, didx, isem):
            pltpu.async_copy(dst_hbm.at[pl.ds(base_of(k), GP)], didx, isem)

        def wait_idx(didx, isem):
            pltpu.make_async_copy(dst_hbm.at[pl.ds(0, GP)], didx, isem).wait()

        start_idx(0, didxA, isemA)

        def body(p, carry):
            k = 2 * p
            wait_idx(didxA, isemA)
            start_idx(k + 1, didxB, isemB)
            pltpu.sync_copy(rows, acc.at[didxA], add=True)
            wait_idx(didxB, isemB)
            start_idx(k + 2, didxA, isemA)
            pltpu.sync_copy(rows, acc.at[didxB], add=True)
            return carry

        lax.fori_loop(0, PAIRS, body, 0)
        wait_idx(didxA, isemA)

        plsc.subcore_barrier()

        for i in range(RPT // GP):
            pltpu.sync_copy(acc.at[pl.ds(rbase + i * GP, GP)], rows)
            pltpu.sync_copy(rows, out_hbm.at[core, pl.ds(rbase + i * GP, GP)])

    return cnt_k(dst, z2d, o2d)


def _tc_dense(xe, agg_part, cnt_part, W_l, W_r, b, relu):
    B = 1280

    def body(x_ref, a_ref, c_ref, wl_ref, wr_ref, b_ref, o_ref):
        a = a_ref[0] + a_ref[1]
        c = c_ref[0] + c_ref[1]
        mean = a / jnp.maximum(c, 1.0)
        dn = (((1,), (1,)), ((), ()))
        out = (lax.dot_general(mean, wl_ref[...], dn,
                               preferred_element_type=jnp.float32)
               + lax.dot_general(x_ref[...], wr_ref[...], dn,
                                 preferred_element_type=jnp.float32)
               + b_ref[...][None, :])
        if relu:
            out = jnp.maximum(out, 0.0)
        o_ref[...] = out

    return pl.pallas_call(
        body,
        grid=(NP // B,),
        in_specs=[
            pl.BlockSpec((B, D), lambda i: (i, 0)),
            pl.BlockSpec((2, B, D), lambda i: (0, i, 0)),
            pl.BlockSpec((2, B, D), lambda i: (0, i, 0)),
            pl.BlockSpec((D, D), lambda i: (0, 0)),
            pl.BlockSpec((D, D), lambda i: (0, 0)),
            pl.BlockSpec((D,), lambda i: (0,)),
        ],
        out_specs=pl.BlockSpec((B, D), lambda i: (i, 0)),
        out_shape=jax.ShapeDtypeStruct((NP, D), jnp.float32),
    )(xe, agg_part, cnt_part, W_l, W_r, b)


def kernel(x, edge_index, W1_l, W1_r, b1, W2_l, W2_r, b2):
    src = edge_index[0]
    dst = edge_index[1]

    npad = E_PAD - N_EDGES
    pad_i = jnp.arange(npad, dtype=jnp.int32)
    srcp = jnp.concatenate([src, pad_i % N_NODES])
    dstp = jnp.concatenate([dst, N_NODES + pad_i % (NP - N_NODES)])
    xe = jnp.pad(x, ((0, NP - N_NODES), (0, 0)))
    z2d = jnp.zeros((GP, D), jnp.float32)
    o2d = jnp.ones((GP, D), jnp.float32)

    cnt = _sc_count(dstp, z2d, o2d)
    agg1 = _sc_aggregate(xe, srcp, dstp, z2d)
    h = _tc_dense(xe, agg1, cnt, W1_l, W1_r, b1, relu=True)
    agg2 = _sc_aggregate(h, srcp, dstp, z2d)
    out = _tc_dense(h, agg2, cnt, W2_l, W2_r, b2, relu=False)
    return out[:N_NODES]


# speedup vs baseline: 2.6002x; 1.2785x over previous
import functools

import jax
import jax.numpy as jnp
from jax import lax
from jax.experimental import pallas as pl
from jax.experimental.pallas import tpu as pltpu
from jax.experimental.pallas import tpu_sc as plsc

N_NODES = 10000
N_EDGES = 320000
D = 128
NP = 10240
NW = 32
GP = 128
NG = 2560
E_PAD = NG * GP
T = NG // NW
PAIRS = T // 2
RPT = NP // 16

_MESH = plsc.VectorSubcoreMesh(core_axis_name="c", subcore_axis_name="s")


KB = 8
NB = T // KB
BPAIRS = NB // 2


KB = 8
NB = T // KB
BPAIRS = NB // 2


def _sc_aggregate(xe, src2, dst2, z2d):
    @functools.partial(
        pl.kernel,
        mesh=_MESH,
        out_type=jax.ShapeDtypeStruct((2, NP, D), jnp.float32),
        scratch_types=[
            pltpu.VMEM((KB, GP), jnp.int32),
            pltpu.VMEM((KB, GP), jnp.int32),
            pltpu.VMEM((KB, GP), jnp.int32),
            pltpu.VMEM((KB, GP), jnp.int32),
            pltpu.VMEM((GP, D), jnp.float32),
            pltpu.VMEM_SHARED((NP, D), jnp.float32),
            pltpu.SemaphoreType.DMA,
            pltpu.SemaphoreType.DMA,
            pltpu.SemaphoreType.DMA,
        ],
    )
    def agg(xe_hbm, src_hbm, dst_hbm, z2d_hbm, out_hbm,
            sidxA, didxA, sidxB, didxB, rows, acc, isemA, isemB, gsem):
        core = lax.axis_index("c")
        tid = lax.axis_index("s")
        w = core * 16 + tid

        pltpu.sync_copy(z2d_hbm, rows)
        rbase = tid * RPT
        for i in range(RPT // GP):
            pltpu.sync_copy(rows, acc.at[pl.ds(rbase + i * GP, GP)])
        plsc.subcore_barrier()

        def start_idx(b, sidx, didx, isem):
            row = (w * NB + jnp.minimum(b, NB - 1)) * KB
            pltpu.async_copy(src_hbm.at[pl.ds(row, KB)], sidx, isem)
            pltpu.async_copy(dst_hbm.at[pl.ds(row, KB)], didx, isem)

        def wait_idx(sidx, didx, isem):
            pltpu.make_async_copy(src_hbm.at[pl.ds(0, KB)], sidx, isem).wait()
            pltpu.make_async_copy(dst_hbm.at[pl.ds(0, KB)], didx, isem).wait()

        def run_batch(sidx, didx):
            for j in range(KB):
                pltpu.async_copy(xe_hbm.at[sidx.at[j]], rows, gsem).wait()
                pltpu.sync_copy(rows, acc.at[didx.at[j]], add=True)

        start_idx(0, sidxA, didxA, isemA)

        def body(p, carry):
            b = 2 * p
            wait_idx(sidxA, didxA, isemA)
            start_idx(b + 1, sidxB, didxB, isemB)
            run_batch(sidxA, didxA)
            wait_idx(sidxB, didxB, isemB)
            start_idx(b + 2, sidxA, didxA, isemA)
            run_batch(sidxB, didxB)
            return carry

        lax.fori_loop(0, BPAIRS, body, 0)
        wait_idx(sidxA, didxA, isemA)

        plsc.subcore_barrier()

        for i in range(RPT // GP):
            pltpu.sync_copy(acc.at[pl.ds(rbase + i * GP, GP)], rows)
            pltpu.sync_copy(rows, out_hbm.at[core, pl.ds(rbase + i * GP, GP)])

    return agg(xe, src2, dst2, z2d)


def _sc_count(dst, z2d, o2d):
    @functools.partial(
        pl.kernel,
        mesh=_MESH,
        out_type=jax.ShapeDtypeStruct((2, NP, D), jnp.float32),
        scratch_types=[
            pltpu.VMEM((GP,), jnp.int32),
            pltpu.VMEM((GP,), jnp.int32),
            pltpu.VMEM((GP, D), jnp.float32),
            pltpu.VMEM_SHARED((NP, D), jnp.float32),
            pltpu.SemaphoreType.DMA,
            pltpu.SemaphoreType.DMA,
        ],
    )
    def cnt_k(dst_hbm, z2d_hbm, o2d_hbm, out_hbm, didxA, didxB, rows, acc,
              isemA, isemB):
        core = lax.axis_index("c")
        tid = lax.axis_index("s")
        w = core * 16 + tid

        pltpu.sync_copy(z2d_hbm, rows)
        rbase = tid * RPT
        for i in range(RPT // GP):
            pltpu.sync_copy(rows, acc.at[pl.ds(rbase + i * GP, GP)])
        plsc.subcore_barrier()

        pltpu.sync_copy(o2d_hbm, rows)

        def base_of(k):
            return jnp.minimum(k * NW + w, NG - 1) * GP

        def start_idx(k, didx, isem):
            pltpu.async_copy(dst_hbm.at[pl.ds(base_of(k), GP)], didx, isem)

        def wait_idx(didx, isem):
            pltpu.make_async_copy(dst_hbm.at[pl.ds(0, GP)], didx, isem).wait()

        start_idx(0, didxA, isemA)

        def body(p, carry):
            k = 2 * p
            wait_idx(didxA, isemA)
            start_idx(k + 1, didxB, isemB)
            pltpu.sync_copy(rows, acc.at[didxA], add=True)
            wait_idx(didxB, isemB)
            start_idx(k + 2, didxA, isemA)
            pltpu.sync_copy(rows, acc.at[didxB], add=True)
            return carry

        lax.fori_loop(0, PAIRS, body, 0)
        wait_idx(didxA, isemA)

        plsc.subcore_barrier()

        for i in range(RPT // GP):
            pltpu.sync_copy(acc.at[pl.ds(rbase + i * GP, GP)], rows)
            pltpu.sync_copy(rows, out_hbm.at[core, pl.ds(rbase + i * GP, GP)])

    return cnt_k(dst, z2d, o2d)


def _tc_dense(xe, agg_part, cnt_part, W_l, W_r, b, relu):
    B = 1280

    def body(x_ref, a_ref, c_ref, wl_ref, wr_ref, b_ref, o_ref):
        a = a_ref[0] + a_ref[1]
        c = c_ref[0] + c_ref[1]
        mean = a / jnp.maximum(c, 1.0)
        dn = (((1,), (1,)), ((), ()))
        out = (lax.dot_general(mean, wl_ref[...], dn,
                               preferred_element_type=jnp.float32)
               + lax.dot_general(x_ref[...], wr_ref[...], dn,
                                 preferred_element_type=jnp.float32)
               + b_ref[...][None, :])
        if relu:
            out = jnp.maximum(out, 0.0)
        o_ref[...] = out

    return pl.pallas_call(
        body,
        grid=(NP // B,),
        in_specs=[
            pl.BlockSpec((B, D), lambda i: (i, 0)),
            pl.BlockSpec((2, B, D), lambda i: (0, i, 0)),
            pl.BlockSpec((2, B, D), lambda i: (0, i, 0)),
            pl.BlockSpec((D, D), lambda i: (0, 0)),
            pl.BlockSpec((D, D), lambda i: (0, 0)),
            pl.BlockSpec((D,), lambda i: (0,)),
        ],
        out_specs=pl.BlockSpec((B, D), lambda i: (i, 0)),
        out_shape=jax.ShapeDtypeStruct((NP, D), jnp.float32),
    )(xe, agg_part, cnt_part, W_l, W_r, b)


def kernel(x, edge_index, W1_l, W1_r, b1, W2_l, W2_r, b2):
    src = edge_index[0]
    dst = edge_index[1]

    npad = E_PAD - N_EDGES
    pad_i = jnp.arange(npad, dtype=jnp.int32)
    srcp = jnp.concatenate([src, pad_i % N_NODES])
    dstp = jnp.concatenate([dst, N_NODES + pad_i % (NP - N_NODES)])
    src2 = srcp.reshape(NG, GP)
    dst2 = dstp.reshape(NG, GP)

    xe = jnp.pad(x, ((0, NP - N_NODES), (0, 0)))
    z2d = jnp.zeros((GP, D), jnp.float32)
    o2d = jnp.ones((GP, D), jnp.float32)

    cnt = _sc_count(dstp, z2d, o2d)
    agg1 = _sc_aggregate(xe, src2, dst2, z2d)
    h = _tc_dense(xe, agg1, cnt, W1_l, W1_r, b1, relu=True)
    agg2 = _sc_aggregate(h, src2, dst2, z2d)
    out = _tc_dense(h, agg2, cnt, W2_l, W2_r, b2, relu=False)
    return out[:N_NODES]


# speedup vs baseline: 3.3663x; 1.2946x over previous
import functools

import jax
import jax.numpy as jnp
from jax import lax
from jax.experimental import pallas as pl
from jax.experimental.pallas import tpu as pltpu
from jax.experimental.pallas import tpu_sc as plsc

N_NODES = 10000
N_EDGES = 320000
D = 128
NP = 10240
NW = 32
GP = 128
NG = 2560
E_PAD = NG * GP
T = NG // NW
PAIRS = T // 2
RPT = NP // 16

_MESH = plsc.VectorSubcoreMesh(core_axis_name="c", subcore_axis_name="s")


KB = 8
NB = T // KB
BPAIRS = NB // 2


KB = 8
NB = T // KB
BPAIRS = NB // 2


def _sc_aggregate(xe, src2, dst2, z2d):
    @functools.partial(
        pl.kernel,
        mesh=_MESH,
        out_type=jax.ShapeDtypeStruct((2, NP, D), jnp.float32),
        scratch_types=[
            pltpu.VMEM((KB, GP), jnp.int32),
            pltpu.VMEM((KB, GP), jnp.int32),
            pltpu.VMEM((KB, GP), jnp.int32),
            pltpu.VMEM((KB, GP), jnp.int32),
            pltpu.VMEM((GP, D), jnp.float32),
            pltpu.VMEM((GP, D), jnp.float32),
            pltpu.VMEM_SHARED((NP, D), jnp.float32),
            pltpu.SemaphoreType.DMA,
            pltpu.SemaphoreType.DMA,
            pltpu.SemaphoreType.DMA,
            pltpu.SemaphoreType.DMA,
        ],
    )
    def agg(xe_hbm, src_hbm, dst_hbm, z2d_hbm, out_hbm,
            sidxA, didxA, sidxB, didxB, rows, rowsB, acc,
            isemA, isemB, gsem, gsemB):
        core = lax.axis_index("c")
        tid = lax.axis_index("s")
        w = core * 16 + tid

        pltpu.sync_copy(z2d_hbm, rows)
        rbase = tid * RPT
        for i in range(RPT // GP):
            pltpu.sync_copy(rows, acc.at[pl.ds(rbase + i * GP, GP)])
        plsc.subcore_barrier()

        def start_idx(b, sidx, didx, isem):
            row = (w * NB + jnp.minimum(b, NB - 1)) * KB
            pltpu.async_copy(src_hbm.at[pl.ds(row, KB)], sidx, isem)
            pltpu.async_copy(dst_hbm.at[pl.ds(row, KB)], didx, isem)

        def wait_idx(sidx, didx, isem):
            pltpu.make_async_copy(src_hbm.at[pl.ds(0, KB)], sidx, isem).wait()
            pltpu.make_async_copy(dst_hbm.at[pl.ds(0, KB)], didx, isem).wait()

        bufs = (rows, rowsB)
        sems = (gsem, gsemB)

        def run_batch(sidx, didx):
            h = pltpu.async_copy(xe_hbm.at[sidx.at[0]], bufs[0], sems[0])
            for j in range(KB):
                if j + 1 < KB:
                    hn = pltpu.async_copy(
                        xe_hbm.at[sidx.at[j + 1]], bufs[(j + 1) % 2],
                        sems[(j + 1) % 2])
                h.wait()
                pltpu.sync_copy(bufs[j % 2], acc.at[didx.at[j]], add=True)
                if j + 1 < KB:
                    h = hn

        start_idx(0, sidxA, didxA, isemA)

        def body(p, carry):
            b = 2 * p
            wait_idx(sidxA, didxA, isemA)
            start_idx(b + 1, sidxB, didxB, isemB)
            run_batch(sidxA, didxA)
            wait_idx(sidxB, didxB, isemB)
            start_idx(b + 2, sidxA, didxA, isemA)
            run_batch(sidxB, didxB)
            return carry

        lax.fori_loop(0, BPAIRS, body, 0)
        wait_idx(sidxA, didxA, isemA)

        plsc.subcore_barrier()

        for i in range(RPT // GP):
            pltpu.sync_copy(acc.at[pl.ds(rbase + i * GP, GP)], rows)
            pltpu.sync_copy(rows, out_hbm.at[core, pl.ds(rbase + i * GP, GP)])

    return agg(xe, src2, dst2, z2d)


def _sc_count(dst, z2d, o2d):
    @functools.partial(
        pl.kernel,
        mesh=_MESH,
        out_type=jax.ShapeDtypeStruct((2, NP, D), jnp.float32),
        scratch_types=[
            pltpu.VMEM((GP,), jnp.int32),
            pltpu.VMEM((GP,), jnp.int32),
            pltpu.VMEM((GP, D), jnp.float32),
            pltpu.VMEM_SHARED((NP, D), jnp.float32),
            pltpu.SemaphoreType.DMA,
            pltpu.SemaphoreType.DMA,
        ],
    )
    def cnt_k(dst_hbm, z2d_hbm, o2d_hbm, out_hbm, didxA, didxB, rows, acc,
              isemA, isemB):
        core = lax.axis_index("c")
        tid = lax.axis_index("s")
        w = core * 16 + tid

        pltpu.sync_copy(z2d_hbm, rows)
        rbase = tid * RPT
        for i in range(RPT // GP):
            pltpu.sync_copy(rows, acc.at[pl.ds(rbase + i * GP, GP)])
        plsc.subcore_barrier()

        pltpu.sync_copy(o2d_hbm, rows)

        def base_of(k):
            return jnp.minimum(k * NW + w, NG - 1) * GP

        def start_idx(k, didx, isem):
            pltpu.async_copy(dst_hbm.at[pl.ds(base_of(k), GP)], didx, isem)

        def wait_idx(didx, isem):
            pltpu.make_async_copy(dst_hbm.at[pl.ds(0, GP)], didx, isem).wait()

        start_idx(0, didxA, isemA)

        def body(p, carry):
            k = 2 * p
            wait_idx(didxA, isemA)
            start_idx(k + 1, didxB, isemB)
            pltpu.sync_copy(rows, acc.at[didxA], add=True)
            wait_idx(didxB, isemB)
            start_idx(k + 2, didxA, isemA)
            pltpu.sync_copy(rows, acc.at[didxB], add=True)
            return carry

        lax.fori_loop(0, PAIRS, body, 0)
        wait_idx(didxA, isemA)

        plsc.subcore_barrier()

        for i in range(RPT // GP):
            pltpu.sync_copy(acc.at[pl.ds(rbase + i * GP, GP)], rows)
            pltpu.sync_copy(rows, out_hbm.at[core, pl.ds(rbase + i * GP, GP)])

    return cnt_k(dst, z2d, o2d)


def _tc_dense(xe, agg_part, cnt_part, W_l, W_r, b, relu):
    B = 1280

    def body(x_ref, a_ref, c_ref, wl_ref, wr_ref, b_ref, o_ref):
        a = a_ref[0] + a_ref[1]
        c = c_ref[0] + c_ref[1]
        mean = a / jnp.maximum(c, 1.0)
        dn = (((1,), (1,)), ((), ()))
        out = (lax.dot_general(mean, wl_ref[...], dn,
                               preferred_element_type=jnp.float32)
               + lax.dot_general(x_ref[...], wr_ref[...], dn,
                                 preferred_element_type=jnp.float32)
               + b_ref[...][None, :])
        if relu:
            out = jnp.maximum(out, 0.0)
        o_ref[...] = out

    return pl.pallas_call(
        body,
        grid=(NP // B,),
        in_specs=[
            pl.BlockSpec((B, D), lambda i: (i, 0)),
            pl.BlockSpec((2, B, D), lambda i: (0, i, 0)),
            pl.BlockSpec((2, B, D), lambda i: (0, i, 0)),
            pl.BlockSpec((D, D), lambda i: (0, 0)),
            pl.BlockSpec((D, D), lambda i: (0, 0)),
            pl.BlockSpec((D,), lambda i: (0,)),
        ],
        out_specs=pl.BlockSpec((B, D), lambda i: (i, 0)),
        out_shape=jax.ShapeDtypeStruct((NP, D), jnp.float32),
    )(xe, agg_part, cnt_part, W_l, W_r, b)


def kernel(x, edge_index, W1_l, W1_r, b1, W2_l, W2_r, b2):
    src = edge_index[0]
    dst = edge_index[1]

    npad = E_PAD - N_EDGES
    pad_i = jnp.arange(npad, dtype=jnp.int32)
    srcp = jnp.concatenate([src, pad_i % N_NODES])
    dstp = jnp.concatenate([dst, N_NODES + pad_i % (NP - N_NODES)])
    src2 = srcp.reshape(NG, GP)
    dst2 = dstp.reshape(NG, GP)

    xe = jnp.pad(x, ((0, NP - N_NODES), (0, 0)))
    z2d = jnp.zeros((GP, D), jnp.float32)
    o2d = jnp.ones((GP, D), jnp.float32)

    cnt = _sc_count(dstp, z2d, o2d)
    agg1 = _sc_aggregate(xe, src2, dst2, z2d)
    h = _tc_dense(xe, agg1, cnt, W1_l, W1_r, b1, relu=True)
    agg2 = _sc_aggregate(h, src2, dst2, z2d)
    out = _tc_dense(h, agg2, cnt, W2_l, W2_r, b2, relu=False)
    return out[:N_NODES]


# speedup vs baseline: 3.4176x; 1.0152x over previous
import functools

import jax
import jax.numpy as jnp
from jax import lax
from jax.experimental import pallas as pl
from jax.experimental.pallas import tpu as pltpu
from jax.experimental.pallas import tpu_sc as plsc

N_NODES = 10000
N_EDGES = 320000
D = 128
NP = 10240
NW = 32
GP = 128
NG = 2560
E_PAD = NG * GP
T = NG // NW
PAIRS = T // 2
RPT = NP // 16

_MESH = plsc.VectorSubcoreMesh(core_axis_name="c", subcore_axis_name="s")


KB = 8
NB = T // KB
BPAIRS = NB // 2


KB = 8
NB = T // KB
BPAIRS = NB // 2


def _sc_aggregate(xe, src2, dst2, z2d):
    @functools.partial(
        pl.kernel,
        mesh=_MESH,
        out_type=jax.ShapeDtypeStruct((2, NP, D), jnp.float32),
        scratch_types=[
            pltpu.VMEM((KB, GP), jnp.int32),
            pltpu.VMEM((KB, GP), jnp.int32),
            pltpu.VMEM((KB, GP), jnp.int32),
            pltpu.VMEM((KB, GP), jnp.int32),
            pltpu.VMEM((GP, D), jnp.float32),
            pltpu.VMEM((GP, D), jnp.float32),
            pltpu.VMEM_SHARED((NP, D), jnp.float32),
            pltpu.SemaphoreType.DMA,
            pltpu.SemaphoreType.DMA,
            pltpu.SemaphoreType.DMA,
            pltpu.SemaphoreType.DMA,
        ],
    )
    def agg(xe_hbm, src_hbm, dst_hbm, z2d_hbm, out_hbm,
            sidxA, didxA, sidxB, didxB, rows, rowsB, acc,
            isemA, isemB, gsem, gsemB):
        core = lax.axis_index("c")
        tid = lax.axis_index("s")
        w = core * 16 + tid

        pltpu.sync_copy(z2d_hbm, rows)
        rbase = tid * RPT
        for i in range(RPT // GP):
            pltpu.sync_copy(rows, acc.at[pl.ds(rbase + i * GP, GP)])
        plsc.subcore_barrier()

        def start_idx(b, sidx, didx, isem):
            row = (w * NB + jnp.minimum(b, NB - 1)) * KB
            pltpu.async_copy(src_hbm.at[pl.ds(row, KB)], sidx, isem)
            pltpu.async_copy(dst_hbm.at[pl.ds(row, KB)], didx, isem)

        def wait_idx(sidx, didx, isem):
            pltpu.make_async_copy(src_hbm.at[pl.ds(0, KB)], sidx, isem).wait()
            pltpu.make_async_copy(dst_hbm.at[pl.ds(0, KB)], didx, isem).wait()

        bufs = (rows, rowsB)
        sems = (gsem, gsemB)

        def run_batch(sidx, didx):
            h = pltpu.async_copy(xe_hbm.at[sidx.at[0]], bufs[0], sems[0])
            for j in range(KB):
                if j + 1 < KB:
                    hn = pltpu.async_copy(
                        xe_hbm.at[sidx.at[j + 1]], bufs[(j + 1) % 2],
                        sems[(j + 1) % 2])
                h.wait()
                pltpu.sync_copy(bufs[j % 2], acc.at[didx.at[j]], add=True)
                if j + 1 < KB:
                    h = hn

        start_idx(0, sidxA, didxA, isemA)

        def body(p, carry):
            b = 2 * p
            wait_idx(sidxA, didxA, isemA)
            start_idx(b + 1, sidxB, didxB, isemB)
            run_batch(sidxA, didxA)
            wait_idx(sidxB, didxB, isemB)
            start_idx(b + 2, sidxA, didxA, isemA)
            run_batch(sidxB, didxB)
            return carry

        lax.fori_loop(0, BPAIRS, body, 0)
        wait_idx(sidxA, didxA, isemA)

        plsc.subcore_barrier()

        for i in range(RPT // GP):
            pltpu.sync_copy(acc.at[pl.ds(rbase + i * GP, GP)], rows)
            pltpu.sync_copy(rows, out_hbm.at[core, pl.ds(rbase + i * GP, GP)])

    return agg(xe, src2, dst2, z2d)


def _sc_count(dst2, z2d, o2d):
    @functools.partial(
        pl.kernel,
        mesh=_MESH,
        out_type=jax.ShapeDtypeStruct((2, NP, D), jnp.float32),
        scratch_types=[
            pltpu.VMEM((KB, GP), jnp.int32),
            pltpu.VMEM((KB, GP), jnp.int32),
            pltpu.VMEM((GP, D), jnp.float32),
            pltpu.VMEM_SHARED((NP, D), jnp.float32),
            pltpu.SemaphoreType.DMA,
            pltpu.SemaphoreType.DMA,
            pltpu.SemaphoreType.DMA,
            pltpu.SemaphoreType.DMA,
        ],
    )
    def cnt_k(dst_hbm, z2d_hbm, o2d_hbm, out_hbm, didxA, didxB, rows, acc,
              isemA, isemB, ssemA, ssemB):
        core = lax.axis_index("c")
        tid = lax.axis_index("s")
        w = core * 16 + tid

        pltpu.sync_copy(z2d_hbm, rows)
        rbase = tid * RPT
        for i in range(RPT // GP):
            pltpu.sync_copy(rows, acc.at[pl.ds(rbase + i * GP, GP)])
        plsc.subcore_barrier()

        pltpu.sync_copy(o2d_hbm, rows)

        def start_idx(b, didx, isem):
            row = (w * NB + jnp.minimum(b, NB - 1)) * KB
            pltpu.async_copy(dst_hbm.at[pl.ds(row, KB)], didx, isem)

        def wait_idx(didx, isem):
            pltpu.make_async_copy(dst_hbm.at[pl.ds(0, KB)], didx, isem).wait()

        def fire_batch(didx, ssem):
            for j in range(KB):
                pltpu.async_copy(rows, acc.at[didx.at[j]], ssem, add=True)

        def drain_batch(didx, ssem):
            for j in range(KB):
                pltpu.make_async_copy(rows, acc.at[didx.at[j]], ssem).wait()

        start_idx(0, didxA, isemA)

        def body(p, carry):
            b = 2 * p
            wait_idx(didxA, isemA)
            start_idx(b + 1, didxB, isemB)
            fire_batch(didxA, ssemA)
            wait_idx(didxB, isemB)
            fire_batch(didxB, ssemB)
            drain_batch(didxA, ssemA)
            start_idx(b + 2, didxA, isemA)
            drain_batch(didxB, ssemB)
            return carry

        lax.fori_loop(0, BPAIRS, body, 0)
        wait_idx(didxA, isemA)

        plsc.subcore_barrier()

        for i in range(RPT // GP):
            pltpu.sync_copy(acc.at[pl.ds(rbase + i * GP, GP)], rows)
            pltpu.sync_copy(rows, out_hbm.at[core, pl.ds(rbase + i * GP, GP)])

    return cnt_k(dst2, z2d, o2d)


def _tc_dense(xe, agg_part, cnt_part, W_l, W_r, b, relu):
    B = 1280

    def body(x_ref, a_ref, c_ref, wl_ref, wr_ref, b_ref, o_ref):
        a = a_ref[0] + a_ref[1]
        c = c_ref[0] + c_ref[1]
        mean = a / jnp.maximum(c, 1.0)
        dn = (((1,), (1,)), ((), ()))
        out = (lax.dot_general(mean, wl_ref[...], dn,
                               preferred_element_type=jnp.float32)
               + lax.dot_general(x_ref[...], wr_ref[...], dn,
                                 preferred_element_type=jnp.float32)
               + b_ref[...][None, :])
        if relu:
            out = jnp.maximum(out, 0.0)
        o_ref[...] = out

    return pl.pallas_call(
        body,
        grid=(NP // B,),
        in_specs=[
            pl.BlockSpec((B, D), lambda i: (i, 0)),
            pl.BlockSpec((2, B, D), lambda i: (0, i, 0)),
            pl.BlockSpec((2, B, D), lambda i: (0, i, 0)),
            pl.BlockSpec((D, D), lambda i: (0, 0)),
            pl.BlockSpec((D, D), lambda i: (0, 0)),
            pl.BlockSpec((D,), lambda i: (0,)),
        ],
        out_specs=pl.BlockSpec((B, D), lambda i: (i, 0)),
        out_shape=jax.ShapeDtypeStruct((NP, D), jnp.float32),
    )(xe, agg_part, cnt_part, W_l, W_r, b)


def kernel(x, edge_index, W1_l, W1_r, b1, W2_l, W2_r, b2):
    src = edge_index[0]
    dst = edge_index[1]

    npad = E_PAD - N_EDGES
    pad_i = jnp.arange(npad, dtype=jnp.int32)
    srcp = jnp.concatenate([src, pad_i % N_NODES])
    dstp = jnp.concatenate([dst, N_NODES + pad_i % (NP - N_NODES)])
    src2 = srcp.reshape(NG, GP)
    dst2 = dstp.reshape(NG, GP)

    xe = jnp.pad(x, ((0, NP - N_NODES), (0, 0)))
    z2d = jnp.zeros((GP, D), jnp.float32)
    o2d = jnp.ones((GP, D), jnp.float32)

    cnt = _sc_count(dst2, z2d, o2d)
    agg1 = _sc_aggregate(xe, src2, dst2, z2d)
    h = _tc_dense(xe, agg1, cnt, W1_l, W1_r, b1, relu=True)
    agg2 = _sc_aggregate(h, src2, dst2, z2d)
    out = _tc_dense(h, agg2, cnt, W2_l, W2_r, b2, relu=False)
    return out[:N_NODES]


# speedup vs baseline: 3.5656x; 1.0433x over previous
import functools

import jax
import jax.numpy as jnp
from jax import lax
from jax.experimental import pallas as pl
from jax.experimental.pallas import tpu as pltpu
from jax.experimental.pallas import tpu_sc as plsc

N_NODES = 10000
N_EDGES = 320000
D = 128
NP = 10240
NW = 32
GP = 128
NG = 2560
E_PAD = NG * GP
T = NG // NW
PAIRS = T // 2
RPT = NP // 16

_MESH = plsc.VectorSubcoreMesh(core_axis_name="c", subcore_axis_name="s")


KB = 8
NB = T // KB
BPAIRS = NB // 2


KB = 8
NB = T // KB
BPAIRS = NB // 2


def _sc_aggregate(xe, src2, dst2, z2d):
    @functools.partial(
        pl.kernel,
        mesh=_MESH,
        out_type=jax.ShapeDtypeStruct((2, NP, D), jnp.float32),
        scratch_types=[
            pltpu.VMEM((KB, GP), jnp.int32),
            pltpu.VMEM((KB, GP), jnp.int32),
            pltpu.VMEM((KB, GP), jnp.int32),
            pltpu.VMEM((KB, GP), jnp.int32),
            pltpu.VMEM((GP, D), jnp.float32),
            pltpu.VMEM((GP, D), jnp.float32),
            pltpu.VMEM_SHARED((NP, D), jnp.float32),
            pltpu.SemaphoreType.DMA,
            pltpu.SemaphoreType.DMA,
            pltpu.SemaphoreType.DMA,
            pltpu.SemaphoreType.DMA,
        ],
    )
    def agg(xe_hbm, src_hbm, dst_hbm, z2d_hbm, out_hbm,
            sidxA, didxA, sidxB, didxB, rows, rowsB, acc,
            isemA, isemB, gsem, gsemB):
        core = lax.axis_index("c")
        tid = lax.axis_index("s")
        w = core * 16 + tid

        def start_idx(b, sidx, didx, isem):
            row = (w * NB + jnp.minimum(b, NB - 1)) * KB
            pltpu.async_copy(src_hbm.at[pl.ds(row, KB)], sidx, isem)
            pltpu.async_copy(dst_hbm.at[pl.ds(row, KB)], didx, isem)

        def wait_idx(sidx, didx, isem):
            pltpu.make_async_copy(src_hbm.at[pl.ds(0, KB)], sidx, isem).wait()
            pltpu.make_async_copy(dst_hbm.at[pl.ds(0, KB)], didx, isem).wait()

        bufs = (rows, rowsB)
        sems = (gsem, gsemB)

        def run_batch(sidx, didx):
            h = pltpu.async_copy(xe_hbm.at[sidx.at[0]], bufs[0], sems[0])
            for j in range(KB):
                if j + 1 < KB:
                    hn = pltpu.async_copy(
                        xe_hbm.at[sidx.at[j + 1]], bufs[(j + 1) % 2],
                        sems[(j + 1) % 2])
                h.wait()
                pltpu.sync_copy(bufs[j % 2], acc.at[didx.at[j]], add=True)
                if j + 1 < KB:
                    h = hn

        pltpu.sync_copy(z2d_hbm, rows)
        rbase = tid * RPT
        zh = [pltpu.async_copy(rows, acc.at[pl.ds(rbase + i * GP, GP)], gsem)
              for i in range(RPT // GP)]
        start_idx(0, sidxA, didxA, isemA)
        for h in zh:
            h.wait()
        plsc.subcore_barrier()

        def body(p, carry):
            b = 2 * p
            wait_idx(sidxA, didxA, isemA)
            start_idx(b + 1, sidxB, didxB, isemB)
            run_batch(sidxA, didxA)
            wait_idx(sidxB, didxB, isemB)
            start_idx(b + 2, sidxA, didxA, isemA)
            run_batch(sidxB, didxB)
            return carry

        lax.fori_loop(0, BPAIRS, body, 0)
        wait_idx(sidxA, didxA, isemA)

        plsc.subcore_barrier()

        ph = [None] * (RPT // GP)
        for i in range(RPT // GP):
            if i >= 2:
                ph[i - 2].wait()
            pltpu.sync_copy(acc.at[pl.ds(rbase + i * GP, GP)], bufs[i % 2])
            ph[i] = pltpu.async_copy(
                bufs[i % 2], out_hbm.at[core, pl.ds(rbase + i * GP, GP)], gsemB)
        ph[RPT // GP - 2].wait()
        ph[RPT // GP - 1].wait()

    return agg(xe, src2, dst2, z2d)


def _sc_count(dst2, z2d, o2d):
    @functools.partial(
        pl.kernel,
        mesh=_MESH,
        out_type=jax.ShapeDtypeStruct((2, NP, D), jnp.float32),
        scratch_types=[
            pltpu.VMEM((KB, GP), jnp.int32),
            pltpu.VMEM((KB, GP), jnp.int32),
            pltpu.VMEM((GP, D), jnp.float32),
            pltpu.VMEM((GP, D), jnp.float32),
            pltpu.VMEM_SHARED((NP, D), jnp.float32),
            pltpu.SemaphoreType.DMA,
            pltpu.SemaphoreType.DMA,
            pltpu.SemaphoreType.DMA,
            pltpu.SemaphoreType.DMA,
        ],
    )
    def cnt_k(dst_hbm, z2d_hbm, o2d_hbm, out_hbm, didxA, didxB, rows, ones,
              acc, isemA, isemB, ssemA, ssemB):
        core = lax.axis_index("c")
        tid = lax.axis_index("s")
        w = core * 16 + tid

        def start_idx(b, didx, isem):
            row = (w * NB + jnp.minimum(b, NB - 1)) * KB
            pltpu.async_copy(dst_hbm.at[pl.ds(row, KB)], didx, isem)

        def wait_idx(didx, isem):
            pltpu.make_async_copy(dst_hbm.at[pl.ds(0, KB)], didx, isem).wait()

        def fire_batch(didx, ssem):
            for j in range(KB):
                pltpu.async_copy(ones, acc.at[didx.at[j]], ssem, add=True)

        def drain_batch(didx, ssem):
            for j in range(KB):
                pltpu.make_async_copy(ones, acc.at[didx.at[j]], ssem).wait()

        pltpu.sync_copy(z2d_hbm, rows)
        rbase = tid * RPT
        zh = [pltpu.async_copy(rows, acc.at[pl.ds(rbase + i * GP, GP)], ssemA)
              for i in range(RPT // GP)]
        pltpu.sync_copy(o2d_hbm, ones)
        start_idx(0, didxA, isemA)
        for h in zh:
            h.wait()
        plsc.subcore_barrier()

        def body(p, carry):
            b = 2 * p
            wait_idx(didxA, isemA)
            start_idx(b + 1, didxB, isemB)
            fire_batch(didxA, ssemA)
            wait_idx(didxB, isemB)
            fire_batch(didxB, ssemB)
            drain_batch(didxA, ssemA)
            start_idx(b + 2, didxA, isemA)
            drain_batch(didxB, ssemB)
            return carry

        lax.fori_loop(0, BPAIRS, body, 0)
        wait_idx(didxA, isemA)

        plsc.subcore_barrier()

        bufs = (rows, ones)
        ph = [None] * (RPT // GP)
        for i in range(RPT // GP):
            if i >= 2:
                ph[i - 2].wait()
            pltpu.sync_copy(acc.at[pl.ds(rbase + i * GP, GP)], bufs[i % 2])
            ph[i] = pltpu.async_copy(
                bufs[i % 2], out_hbm.at[core, pl.ds(rbase + i * GP, GP)], ssemB)
        ph[RPT // GP - 2].wait()
        ph[RPT // GP - 1].wait()

    return cnt_k(dst2, z2d, o2d)


def _tc_dense(xe, agg_part, cnt_part, W_l, W_r, b, relu):
    B = 2000

    def body(x_ref, a_ref, c_ref, wl_ref, wr_ref, b_ref, o_ref):
        a = a_ref[0] + a_ref[1]
        c = c_ref[0] + c_ref[1]
        mean = a / jnp.maximum(c, 1.0)
        dn = (((1,), (1,)), ((), ()))
        out = (lax.dot_general(mean, wl_ref[...], dn,
                               preferred_element_type=jnp.float32)
               + lax.dot_general(x_ref[...], wr_ref[...], dn,
                                 preferred_element_type=jnp.float32)
               + b_ref[...][None, :])
        if relu:
            out = jnp.maximum(out, 0.0)
        o_ref[...] = out

    return pl.pallas_call(
        body,
        grid=(N_NODES // B,),
        in_specs=[
            pl.BlockSpec((B, D), lambda i: (i, 0)),
            pl.BlockSpec((2, B, D), lambda i: (0, i, 0)),
            pl.BlockSpec((2, B, D), lambda i: (0, i, 0)),
            pl.BlockSpec((D, D), lambda i: (0, 0)),
            pl.BlockSpec((D, D), lambda i: (0, 0)),
            pl.BlockSpec((D,), lambda i: (0,)),
        ],
        out_specs=pl.BlockSpec((B, D), lambda i: (i, 0)),
        out_shape=jax.ShapeDtypeStruct((N_NODES, D), jnp.float32),
    )(xe, agg_part, cnt_part, W_l, W_r, b)


def kernel(x, edge_index, W1_l, W1_r, b1, W2_l, W2_r, b2):
    src = edge_index[0]
    dst = edge_index[1]

    npad = E_PAD - N_EDGES
    pad_i = jnp.arange(npad, dtype=jnp.int32)
    srcp = jnp.concatenate([src, pad_i % N_NODES])
    dstp = jnp.concatenate([dst, N_NODES + pad_i % (NP - N_NODES)])
    src2 = srcp.reshape(NG, GP)
    dst2 = dstp.reshape(NG, GP)

    z2d = jnp.zeros((GP, D), jnp.float32)
    o2d = jnp.ones((GP, D), jnp.float32)

    cnt = _sc_count(dst2, z2d, o2d)
    agg1 = _sc_aggregate(x, src2, dst2, z2d)
    h = _tc_dense(x, agg1, cnt, W1_l, W1_r, b1, relu=True)
    agg2 = _sc_aggregate(h, src2, dst2, z2d)
    return _tc_dense(h, agg2, cnt, W2_l, W2_r, b2, relu=False)


# speedup vs baseline: 3.5677x; 1.0006x over previous
import functools

import jax
import jax.numpy as jnp
from jax import lax
from jax.experimental import pallas as pl
from jax.experimental.pallas import tpu as pltpu
from jax.experimental.pallas import tpu_sc as plsc

N_NODES = 10000
N_EDGES = 320000
D = 128
NP = 10240
NW = 32
GP = 128
NG = 2560
E_PAD = NG * GP
T = NG // NW
PAIRS = T // 2
RPT = NP // 16

_MESH = plsc.VectorSubcoreMesh(core_axis_name="c", subcore_axis_name="s")


KB = 8
NB = T // KB
BPAIRS = NB // 2


KB = 8
NB = T // KB
BPAIRS = NB // 2


def _sc_aggregate(xe, src2, dst2, z2d):
    @functools.partial(
        pl.kernel,
        mesh=_MESH,
        out_type=jax.ShapeDtypeStruct((2, NP, D), jnp.float32),
        scratch_types=[
            pltpu.VMEM((KB, GP), jnp.int32),
            pltpu.VMEM((KB, GP), jnp.int32),
            pltpu.VMEM((KB, GP), jnp.int32),
            pltpu.VMEM((KB, GP), jnp.int32),
            pltpu.VMEM((GP, D), jnp.float32),
            pltpu.VMEM((GP, D), jnp.float32),
            pltpu.VMEM_SHARED((NP, D), jnp.float32),
            pltpu.SemaphoreType.DMA,
            pltpu.SemaphoreType.DMA,
            pltpu.SemaphoreType.DMA,
            pltpu.SemaphoreType.DMA,
            pltpu.SemaphoreType.DMA,
            pltpu.SemaphoreType.DMA,
        ],
    )
    def agg(xe_hbm, src_hbm, dst_hbm, z2d_hbm, out_hbm,
            sidxA, didxA, sidxB, didxB, rows, rowsB, acc,
            isemA, isemB, gsem, gsemB, ssem, ssemB):
        core = lax.axis_index("c")
        tid = lax.axis_index("s")
        w = core * 16 + tid

        def start_idx(b, sidx, didx, isem):
            row = (w * NB + jnp.minimum(b, NB - 1)) * KB
            pltpu.async_copy(src_hbm.at[pl.ds(row, KB)], sidx, isem)
            pltpu.async_copy(dst_hbm.at[pl.ds(row, KB)], didx, isem)

        def wait_idx(sidx, didx, isem):
            pltpu.make_async_copy(src_hbm.at[pl.ds(0, KB)], sidx, isem).wait()
            pltpu.make_async_copy(dst_hbm.at[pl.ds(0, KB)], didx, isem).wait()

        bufs = (rows, rowsB)
        sems = (gsem, gsemB)
        ssems = (ssem, ssemB)

        def run_batch(sidx, didx):
            h = [None] * KB
            sh = [None] * KB
            h[0] = pltpu.async_copy(xe_hbm.at[sidx.at[0]], bufs[0], sems[0])
            for j in range(KB):
                if j + 1 < KB:
                    if j >= 1:
                        sh[j - 1].wait()
                    h[j + 1] = pltpu.async_copy(
                        xe_hbm.at[sidx.at[j + 1]], bufs[(j + 1) % 2],
                        sems[(j + 1) % 2])
                h[j].wait()
                sh[j] = pltpu.async_copy(bufs[j % 2], acc.at[didx.at[j]],
                                         ssems[j % 2], add=True)
            sh[KB - 2].wait()
            sh[KB - 1].wait()

        pltpu.sync_copy(z2d_hbm, rows)
        rbase = tid * RPT
        zh = [pltpu.async_copy(rows, acc.at[pl.ds(rbase + i * GP, GP)], gsem)
              for i in range(RPT // GP)]
        start_idx(0, sidxA, didxA, isemA)
        for h in zh:
            h.wait()
        plsc.subcore_barrier()

        def body(p, carry):
            b = 2 * p
            wait_idx(sidxA, didxA, isemA)
            start_idx(b + 1, sidxB, didxB, isemB)
            run_batch(sidxA, didxA)
            wait_idx(sidxB, didxB, isemB)
            start_idx(b + 2, sidxA, didxA, isemA)
            run_batch(sidxB, didxB)
            return carry

        lax.fori_loop(0, BPAIRS, body, 0)
        wait_idx(sidxA, didxA, isemA)

        plsc.subcore_barrier()

        ph = [None] * (RPT // GP)
        for i in range(RPT // GP):
            if i >= 2:
                ph[i - 2].wait()
            pltpu.sync_copy(acc.at[pl.ds(rbase + i * GP, GP)], bufs[i % 2])
            ph[i] = pltpu.async_copy(
                bufs[i % 2], out_hbm.at[core, pl.ds(rbase + i * GP, GP)], gsemB)
        ph[RPT // GP - 2].wait()
        ph[RPT // GP - 1].wait()

    return agg(xe, src2, dst2, z2d)


def _sc_count(dst2, z2d, o2d):
    @functools.partial(
        pl.kernel,
        mesh=_MESH,
        out_type=jax.ShapeDtypeStruct((2, NP, D), jnp.float32),
        scratch_types=[
            pltpu.VMEM((KB, GP), jnp.int32),
            pltpu.VMEM((KB, GP), jnp.int32),
            pltpu.VMEM((GP, D), jnp.float32),
            pltpu.VMEM((GP, D), jnp.float32),
            pltpu.VMEM_SHARED((NP, D), jnp.float32),
            pltpu.SemaphoreType.DMA,
            pltpu.SemaphoreType.DMA,
            pltpu.SemaphoreType.DMA,
            pltpu.SemaphoreType.DMA,
        ],
    )
    def cnt_k(dst_hbm, z2d_hbm, o2d_hbm, out_hbm, didxA, didxB, rows, ones,
              acc, isemA, isemB, ssemA, ssemB):
        core = lax.axis_index("c")
        tid = lax.axis_index("s")
        w = core * 16 + tid

        def start_idx(b, didx, isem):
            row = (w * NB + jnp.minimum(b, NB - 1)) * KB
            pltpu.async_copy(dst_hbm.at[pl.ds(row, KB)], didx, isem)

        def wait_idx(didx, isem):
            pltpu.make_async_copy(dst_hbm.at[pl.ds(0, KB)], didx, isem).wait()

        def fire_batch(didx, ssem):
            for j in range(KB):
                pltpu.async_copy(ones, acc.at[didx.at[j]], ssem, add=True)

        def drain_batch(didx, ssem):
            for j in range(KB):
                pltpu.make_async_copy(ones, acc.at[didx.at[j]], ssem).wait()

        pltpu.sync_copy(z2d_hbm, rows)
        rbase = tid * RPT
        zh = [pltpu.async_copy(rows, acc.at[pl.ds(rbase + i * GP, GP)], ssemA)
              for i in range(RPT // GP)]
        pltpu.sync_copy(o2d_hbm, ones)
        start_idx(0, didxA, isemA)
        for h in zh:
            h.wait()
        plsc.subcore_barrier()

        def body(p, carry):
            b = 2 * p
            wait_idx(didxA, isemA)
            start_idx(b + 1, didxB, isemB)
            fire_batch(didxA, ssemA)
            wait_idx(didxB, isemB)
            fire_batch(didxB, ssemB)
            drain_batch(didxA, ssemA)
            start_idx(b + 2, didxA, isemA)
            drain_batch(didxB, ssemB)
            return carry

        lax.fori_loop(0, BPAIRS, body, 0)
        wait_idx(didxA, isemA)

        plsc.subcore_barrier()

        bufs = (rows, ones)
        ph = [None] * (RPT // GP)
        for i in range(RPT // GP):
            if i >= 2:
                ph[i - 2].wait()
            pltpu.sync_copy(acc.at[pl.ds(rbase + i * GP, GP)], bufs[i % 2])
            ph[i] = pltpu.async_copy(
                bufs[i % 2], out_hbm.at[core, pl.ds(rbase + i * GP, GP)], ssemB)
        ph[RPT // GP - 2].wait()
        ph[RPT // GP - 1].wait()

    return cnt_k(dst2, z2d, o2d)


def _tc_dense(xe, agg_part, cnt_part, W_l, W_r, b, relu):
    B = 2000

    def body(x_ref, a_ref, c_ref, wl_ref, wr_ref, b_ref, o_ref):
        a = a_ref[0] + a_ref[1]
        c = c_ref[0] + c_ref[1]
        mean = a / jnp.maximum(c, 1.0)
        dn = (((1,), (1,)), ((), ()))
        out = (lax.dot_general(mean, wl_ref[...], dn,
                               preferred_element_type=jnp.float32)
               + lax.dot_general(x_ref[...], wr_ref[...], dn,
                                 preferred_element_type=jnp.float32)
               + b_ref[...][None, :])
        if relu:
            out = jnp.maximum(out, 0.0)
        o_ref[...] = out

    return pl.pallas_call(
        body,
        grid=(N_NODES // B,),
        in_specs=[
            pl.BlockSpec((B, D), lambda i: (i, 0)),
            pl.BlockSpec((2, B, D), lambda i: (0, i, 0)),
            pl.BlockSpec((2, B, D), lambda i: (0, i, 0)),
            pl.BlockSpec((D, D), lambda i: (0, 0)),
            pl.BlockSpec((D, D), lambda i: (0, 0)),
            pl.BlockSpec((D,), lambda i: (0,)),
        ],
        out_specs=pl.BlockSpec((B, D), lambda i: (i, 0)),
        out_shape=jax.ShapeDtypeStruct((N_NODES, D), jnp.float32),
    )(xe, agg_part, cnt_part, W_l, W_r, b)


def kernel(x, edge_index, W1_l, W1_r, b1, W2_l, W2_r, b2):
    src = edge_index[0]
    dst = edge_index[1]

    npad = E_PAD - N_EDGES
    pad_i = jnp.arange(npad, dtype=jnp.int32)
    srcp = jnp.concatenate([src, pad_i % N_NODES])
    dstp = jnp.concatenate([dst, N_NODES + pad_i % (NP - N_NODES)])
    src2 = srcp.reshape(NG, GP)
    dst2 = dstp.reshape(NG, GP)

    z2d = jnp.zeros((GP, D), jnp.float32)
    o2d = jnp.ones((GP, D), jnp.float32)

    cnt = _sc_count(dst2, z2d, o2d)
    agg1 = _sc_aggregate(x, src2, dst2, z2d)
    h = _tc_dense(x, agg1, cnt, W1_l, W1_r, b1, relu=True)
    agg2 = _sc_aggregate(h, src2, dst2, z2d)
    return _tc_dense(h, agg2, cnt, W2_l, W2_r, b2, relu=False)
